# baseline (speedup 1.0000x reference)
import functools

import jax
import jax.numpy as jnp
from jax.experimental import pallas as pl
from jax.experimental.pallas import tpu as pltpu

_BETA = 1.0
_GAMMA = 0.25
_NSUB = 2


def _gelu(x):
    return 0.5 * x * (1.0 + jax.lax.erf(x * (2.0 ** -0.5)))


def _dot(a, b):
    return jax.lax.dot_general(a, b, (((1,), (0,)), ((), ())),
                               preferred_element_type=jnp.float32)


def _dot_t(a, b):
    return jax.lax.dot_general(a, b, (((1,), (1,)), ((), ())),
                               preferred_element_type=jnp.float32)


def _vq_round(r, C, Cm2, cn):
    s = _dot_t(r, Cm2) + cn
    smin = jnp.min(s, axis=1, keepdims=True)
    oh = (s == smin).astype(jnp.float32)
    q = _dot(oh, C)
    diff = r - q
    return diff, jnp.sum(diff * diff)


def _body(x_ref, w0_ref, b0_ref, w1_ref, b1_ref,
          dw0_ref, db0_ref, dw1_ref, db1_ref,
          cb0_ref, cb1_ref, cb2_ref,
          cm0_ref, cm1_ref, cm2_ref,
          recon_ref, l0_ref, l1_ref, l2_ref,
          cn0_ref, cn1_ref, cn2_ref):
    i = pl.program_id(0)

    @pl.when(i == 0)
    def _init():
        recon_ref[0, 0] = 0.0
        l0_ref[0, 0] = 0.0
        l1_ref[0, 0] = 0.0
        l2_ref[0, 0] = 0.0
        for cref, nref in ((cb0_ref, cn0_ref), (cb1_ref, cn1_ref),
                           (cb2_ref, cn2_ref)):
            C = cref[...]
            nref[...] = jnp.sum(C * C, axis=1)[None, :]

    TB = x_ref.shape[0]
    SUB = TB // _NSUB
    sr = s0 = s1 = s2 = jnp.float32(0.0)
    for sub in range(_NSUB):
        x = x_ref[pl.ds(sub * SUB, SUB), :]
        h = _gelu(_dot(x, w0_ref[...]) + b0_ref[...])
        z = _dot(h, w1_ref[...]) + b1_ref[...]

        r = z
        r, a0 = _vq_round(r, cb0_ref[...], cm0_ref[...], cn0_ref[...])
        r, a1 = _vq_round(r, cb1_ref[...], cm1_ref[...], cn1_ref[...])
        r, a2 = _vq_round(r, cb2_ref[...], cm2_ref[...], cn2_ref[...])
        zq = z - r

        h2 = _gelu(_dot(zq, dw0_ref[...]) + db0_ref[...])
        y = _dot(h2, dw1_ref[...]) + db1_ref[...]
        e = y - x
        sr += jnp.sum(e * e)
        s0, s1, s2 = s0 + a0, s1 + a1, s2 + a2

    recon_ref[0, 0] += sr
    l0_ref[0, 0] += s0
    l1_ref[0, 0] += s1
    l2_ref[0, 0] += s2


@functools.partial(jax.jit, static_argnames=())
def kernel(input, enc_W0, enc_b0, enc_W1, enc_b1,
           dec_W0, dec_b0, dec_W1, dec_b1, cb0, cb1, cb2):
    B, D = input.shape
    Z = enc_W1.shape[1]
    TB = 1024
    grid = B // TB

    full = lambda s: pl.BlockSpec(s, lambda i: (0,) * len(s))
    scal = pl.BlockSpec(memory_space=pltpu.SMEM)

    outs = pl.pallas_call(
        _body,
        grid=(grid,),
        in_specs=[
            pl.BlockSpec((TB, D), lambda i: (i, 0)),
            full(enc_W0.shape), full((1, enc_b0.shape[0])),
            full(enc_W1.shape), full((1, enc_b1.shape[0])),
            full(dec_W0.shape), full((1, dec_b0.shape[0])),
            full(dec_W1.shape), full((1, dec_b1.shape[0])),
            full(cb0.shape), full(cb1.shape), full(cb2.shape),
            full(cb0.shape), full(cb1.shape), full(cb2.shape),
        ],
        out_specs=[scal, scal, scal, scal],
        out_shape=[jax.ShapeDtypeStruct((1, 1), jnp.float32)] * 4,
        scratch_shapes=[
            pltpu.VMEM((1, cb0.shape[0]), jnp.float32),
            pltpu.VMEM((1, cb1.shape[0]), jnp.float32),
            pltpu.VMEM((1, cb2.shape[0]), jnp.float32),
        ],
    )(input, enc_W0, enc_b0[None, :], enc_W1, enc_b1[None, :],
      dec_W0, dec_b0[None, :], dec_W1, dec_b1[None, :], cb0, cb1, cb2,
      -2.0 * cb0, -2.0 * cb1, -2.0 * cb2)

    recon_s, l0_s, l1_s, l2_s = outs
    recon = recon_s[0, 0] / (B * D)
    scale = (_BETA + _GAMMA) / (B * Z)
    return (recon, l0_s[0, 0] * scale, l1_s[0, 0] * scale, l2_s[0, 0] * scale)


# speedup vs baseline: 1.1075x; 1.1075x over previous
import functools

import jax
import jax.numpy as jnp
from jax.experimental import pallas as pl
from jax.experimental.pallas import tpu as pltpu

_BETA = 1.0
_GAMMA = 0.25
_NSUB = 1


def _gelu(x):
    return 0.5 * x * (1.0 + jax.lax.erf(x * (2.0 ** -0.5)))


def _dot(a, b):
    return jax.lax.dot_general(a, b, (((1,), (0,)), ((), ())),
                               preferred_element_type=jnp.float32)


def _dot_t(a, b):
    return jax.lax.dot_general(a, b, (((1,), (1,)), ((), ())),
                               preferred_element_type=jnp.float32)


def _vq_round(r, Cm2, cn):
    s = _dot_t(r, Cm2) + cn
    smin = jnp.min(s, axis=1, keepdims=True)
    oh = (s == smin).astype(jnp.float32)
    q = _dot(oh, Cm2)
    diff = r + 0.5 * q
    return diff, jnp.sum(diff * diff)


def _body(x_ref, w0_ref, b0_ref, w1_ref, b1_ref,
          dw0_ref, db0_ref, dw1_ref, db1_ref,
          cm0_ref, cm1_ref, cm2_ref,
          recon_ref, l0_ref, l1_ref, l2_ref,
          cn0_ref, cn1_ref, cn2_ref):
    i = pl.program_id(0)

    @pl.when(i == 0)
    def _init():
        recon_ref[0, 0] = 0.0
        l0_ref[0, 0] = 0.0
        l1_ref[0, 0] = 0.0
        l2_ref[0, 0] = 0.0
        for cref, nref in ((cm0_ref, cn0_ref), (cm1_ref, cn1_ref),
                           (cm2_ref, cn2_ref)):
            C = cref[...]
            nref[...] = 0.25 * jnp.sum(C * C, axis=1)[None, :]

    TB = x_ref.shape[0]
    SUB = TB // _NSUB
    sr = s0 = s1 = s2 = jnp.float32(0.0)
    for sub in range(_NSUB):
        x = x_ref[pl.ds(sub * SUB, SUB), :]
        h = _gelu(_dot(x, w0_ref[...]) + b0_ref[...])
        z = _dot(h, w1_ref[...]) + b1_ref[...]

        r = z
        r, a0 = _vq_round(r, cm0_ref[...], cn0_ref[...])
        r, a1 = _vq_round(r, cm1_ref[...], cn1_ref[...])
        r, a2 = _vq_round(r, cm2_ref[...], cn2_ref[...])
        zq = z - r

        h2 = _gelu(_dot(zq, dw0_ref[...]) + db0_ref[...])
        y = _dot(h2, dw1_ref[...]) + db1_ref[...]
        e = y - x
        sr += jnp.sum(e * e)
        s0, s1, s2 = s0 + a0, s1 + a1, s2 + a2

    recon_ref[0, 0] += sr
    l0_ref[0, 0] += s0
    l1_ref[0, 0] += s1
    l2_ref[0, 0] += s2


@functools.partial(jax.jit, static_argnames=())
def kernel(input, enc_W0, enc_b0, enc_W1, enc_b1,
           dec_W0, dec_b0, dec_W1, dec_b1, cb0, cb1, cb2):
    B, D = input.shape
    Z = enc_W1.shape[1]
    TB = 1024
    grid = B // TB

    full = lambda s: pl.BlockSpec(s, lambda i: (0,) * len(s))
    scal = pl.BlockSpec(memory_space=pltpu.SMEM)

    outs = pl.pallas_call(
        _body,
        grid=(grid,),
        in_specs=[
            pl.BlockSpec((TB, D), lambda i: (i, 0)),
            full(enc_W0.shape), full((1, enc_b0.shape[0])),
            full(enc_W1.shape), full((1, enc_b1.shape[0])),
            full(dec_W0.shape), full((1, dec_b0.shape[0])),
            full(dec_W1.shape), full((1, dec_b1.shape[0])),
            full(cb0.shape), full(cb1.shape), full(cb2.shape),
        ],
        out_specs=[scal, scal, scal, scal],
        out_shape=[jax.ShapeDtypeStruct((1, 1), jnp.float32)] * 4,
        scratch_shapes=[
            pltpu.VMEM((1, cb0.shape[0]), jnp.float32),
            pltpu.VMEM((1, cb1.shape[0]), jnp.float32),
            pltpu.VMEM((1, cb2.shape[0]), jnp.float32),
        ],
    )(input, enc_W0, enc_b0[None, :], enc_W1, enc_b1[None, :],
      dec_W0, dec_b0[None, :], dec_W1, dec_b1[None, :],
      -2.0 * cb0, -2.0 * cb1, -2.0 * cb2)

    recon_s, l0_s, l1_s, l2_s = outs
    recon = recon_s[0, 0] / (B * D)
    scale = (_BETA + _GAMMA) / (B * Z)
    return (recon, l0_s[0, 0] * scale, l1_s[0, 0] * scale, l2_s[0, 0] * scale)


# speedup vs baseline: 1.1506x; 1.0389x over previous
import functools

import jax
import jax.numpy as jnp
from jax.experimental import pallas as pl
from jax.experimental.pallas import tpu as pltpu

_BETA = 1.0
_GAMMA = 0.25
_NSUB = 1


def _gelu(x):
    return 0.5 * x * (1.0 + jax.lax.erf(x * (2.0 ** -0.5)))


def _dot(a, b):
    return jax.lax.dot_general(a, b, (((1,), (0,)), ((), ())),
                               preferred_element_type=jnp.float32)


def _dot_t(a, b):
    return jax.lax.dot_general(a, b, (((1,), (1,)), ((), ())),
                               preferred_element_type=jnp.float32)


def _vq_round(r, Cm2, cn):
    s = cn - 2.0 * _dot_t(r, Cm2)
    smin = jnp.min(s, axis=1, keepdims=True)
    oh = (s == smin).astype(jnp.float32)
    q = _dot(oh, Cm2)
    diff = r - q
    return diff, jnp.sum(diff * diff)


def _body(x_ref, w0_ref, b0_ref, w1_ref, b1_ref,
          dw0_ref, db0_ref, dw1_ref, db1_ref,
          cm0_ref, cm1_ref, cm2_ref,
          recon_ref, l0_ref, l1_ref, l2_ref,
          cn0_ref, cn1_ref, cn2_ref):
    i = pl.program_id(0)

    @pl.when(i == 0)
    def _init():
        recon_ref[0, 0] = 0.0
        l0_ref[0, 0] = 0.0
        l1_ref[0, 0] = 0.0
        l2_ref[0, 0] = 0.0
        for cref, nref in ((cm0_ref, cn0_ref), (cm1_ref, cn1_ref),
                           (cm2_ref, cn2_ref)):
            C = cref[...]
            nref[...] = jnp.sum(C * C, axis=1)[None, :]

    TB = x_ref.shape[0]
    SUB = TB // _NSUB
    sr = s0 = s1 = s2 = jnp.float32(0.0)
    for sub in range(_NSUB):
        x = x_ref[pl.ds(sub * SUB, SUB), :]
        h = _gelu(_dot(x, w0_ref[...]) + b0_ref[...])
        z = _dot(h, w1_ref[...]) + b1_ref[...]

        r = z
        r, a0 = _vq_round(r, cm0_ref[...], cn0_ref[...])
        r, a1 = _vq_round(r, cm1_ref[...], cn1_ref[...])
        r, a2 = _vq_round(r, cm2_ref[...], cn2_ref[...])
        zq = z - r

        h2 = _gelu(_dot(zq, dw0_ref[...]) + db0_ref[...])
        y = _dot(h2, dw1_ref[...]) + db1_ref[...]
        e = y - x
        sr += jnp.sum(e * e)
        s0, s1, s2 = s0 + a0, s1 + a1, s2 + a2

    recon_ref[0, 0] += sr
    l0_ref[0, 0] += s0
    l1_ref[0, 0] += s1
    l2_ref[0, 0] += s2


@functools.partial(jax.jit, static_argnames=())
def kernel(input, enc_W0, enc_b0, enc_W1, enc_b1,
           dec_W0, dec_b0, dec_W1, dec_b1, cb0, cb1, cb2):
    B, D = input.shape
    Z = enc_W1.shape[1]
    TB = 1024
    grid = B // TB

    full = lambda s: pl.BlockSpec(s, lambda i: (0,) * len(s))
    scal = pl.BlockSpec(memory_space=pltpu.SMEM)

    outs = pl.pallas_call(
        _body,
        grid=(grid,),
        in_specs=[
            pl.BlockSpec((TB, D), lambda i: (i, 0)),
            full(enc_W0.shape), full((1, enc_b0.shape[0])),
            full(enc_W1.shape), full((1, enc_b1.shape[0])),
            full(dec_W0.shape), full((1, dec_b0.shape[0])),
            full(dec_W1.shape), full((1, dec_b1.shape[0])),
            full(cb0.shape), full(cb1.shape), full(cb2.shape),
        ],
        out_specs=[scal, scal, scal, scal],
        out_shape=[jax.ShapeDtypeStruct((1, 1), jnp.float32)] * 4,
        scratch_shapes=[
            pltpu.VMEM((1, cb0.shape[0]), jnp.float32),
            pltpu.VMEM((1, cb1.shape[0]), jnp.float32),
            pltpu.VMEM((1, cb2.shape[0]), jnp.float32),
        ],
    )(input, enc_W0, enc_b0[None, :], enc_W1, enc_b1[None, :],
      dec_W0, dec_b0[None, :], dec_W1, dec_b1[None, :],
      cb0, cb1, cb2)

    recon_s, l0_s, l1_s, l2_s = outs
    recon = recon_s[0, 0] / (B * D)
    scale = (_BETA + _GAMMA) / (B * Z)
    return (recon, l0_s[0, 0] * scale, l1_s[0, 0] * scale, l2_s[0, 0] * scale)


# speedup vs baseline: 1.2251x; 1.0647x over previous
import functools

import jax
import jax.numpy as jnp
from jax.experimental import pallas as pl
from jax.experimental.pallas import tpu as pltpu

_BETA = 1.0
_GAMMA = 0.25
_NSUB = 1


def _gelu(x):
    return 0.5 * x * (1.0 + jax.lax.erf(x * (2.0 ** -0.5)))


def _dot(a, b):
    return jax.lax.dot_general(a, b, (((1,), (0,)), ((), ())),
                               preferred_element_type=jnp.float32)


def _dot_t(a, b):
    return jax.lax.dot_general(a, b, (((1,), (1,)), ((), ())),
                               preferred_element_type=jnp.float32)


def _vq_round(r, Cm2, cn):
    s = cn - 2.0 * _dot_t(r, Cm2)
    smin = jnp.min(s, axis=1, keepdims=True)
    oh = (s == smin).astype(jnp.float32)
    q = _dot(oh, Cm2)
    diff = r - q
    return diff, jnp.sum(diff * diff)


def _body(x_ref, w0_ref, b0_ref, w1_ref, b1_ref,
          dw0_ref, db0_ref, dw1_ref, db1_ref,
          cm0_ref, cm1_ref, cm2_ref,
          recon_ref, l0_ref, l1_ref, l2_ref,
          cn0_ref, cn1_ref, cn2_ref):
    i = pl.program_id(0)

    @pl.when(i == 0)
    def _init():
        recon_ref[0, 0] = 0.0
        l0_ref[0, 0] = 0.0
        l1_ref[0, 0] = 0.0
        l2_ref[0, 0] = 0.0
        for cref, nref in ((cm0_ref, cn0_ref), (cm1_ref, cn1_ref),
                           (cm2_ref, cn2_ref)):
            C = cref[...]
            nref[...] = jnp.sum(C * C, axis=1)[None, :]

    TB = x_ref.shape[0]
    SUB = TB // _NSUB
    sr = s0 = s1 = s2 = jnp.float32(0.0)
    for sub in range(_NSUB):
        x = x_ref[pl.ds(sub * SUB, SUB), :]
        h = _gelu(_dot(x, w0_ref[...]) + b0_ref[...])
        z = _dot(h, w1_ref[...]) + b1_ref[...]

        r = z
        r, a0 = _vq_round(r, cm0_ref[...], cn0_ref[...])
        r, a1 = _vq_round(r, cm1_ref[...], cn1_ref[...])
        r, a2 = _vq_round(r, cm2_ref[...], cn2_ref[...])
        zq = z - r

        h2 = _gelu(_dot(zq, dw0_ref[...]) + db0_ref[...])
        y = _dot(h2, dw1_ref[...]) + db1_ref[...]
        e = y - x
        sr += jnp.sum(e * e)
        s0, s1, s2 = s0 + a0, s1 + a1, s2 + a2

    recon_ref[0, 0] += sr
    l0_ref[0, 0] += s0
    l1_ref[0, 0] += s1
    l2_ref[0, 0] += s2


@functools.partial(jax.jit, static_argnames=())
def kernel(input, enc_W0, enc_b0, enc_W1, enc_b1,
           dec_W0, dec_b0, dec_W1, dec_b1, cb0, cb1, cb2):
    B, D = input.shape
    Z = enc_W1.shape[1]
    TB = 2048
    grid = B // TB

    full = lambda s: pl.BlockSpec(s, lambda i: (0,) * len(s))
    scal = pl.BlockSpec(memory_space=pltpu.SMEM)

    outs = pl.pallas_call(
        _body,
        grid=(grid,),
        in_specs=[
            pl.BlockSpec((TB, D), lambda i: (i, 0)),
            full(enc_W0.shape), full((1, enc_b0.shape[0])),
            full(enc_W1.shape), full((1, enc_b1.shape[0])),
            full(dec_W0.shape), full((1, dec_b0.shape[0])),
            full(dec_W1.shape), full((1, dec_b1.shape[0])),
            full(cb0.shape), full(cb1.shape), full(cb2.shape),
        ],
        out_specs=[scal, scal, scal, scal],
        out_shape=[jax.ShapeDtypeStruct((1, 1), jnp.float32)] * 4,
        scratch_shapes=[
            pltpu.VMEM((1, cb0.shape[0]), jnp.float32),
            pltpu.VMEM((1, cb1.shape[0]), jnp.float32),
            pltpu.VMEM((1, cb2.shape[0]), jnp.float32),
        ],
    )(input, enc_W0, enc_b0[None, :], enc_W1, enc_b1[None, :],
      dec_W0, dec_b0[None, :], dec_W1, dec_b1[None, :],
      cb0, cb1, cb2)

    recon_s, l0_s, l1_s, l2_s = outs
    recon = recon_s[0, 0] / (B * D)
    scale = (_BETA + _GAMMA) / (B * Z)
    return (recon, l0_s[0, 0] * scale, l1_s[0, 0] * scale, l2_s[0, 0] * scale)


# speedup vs baseline: 1.2320x; 1.0056x over previous
import functools

import jax
import jax.numpy as jnp
from jax.experimental import pallas as pl
from jax.experimental.pallas import tpu as pltpu

_BETA = 1.0
_GAMMA = 0.25
_NSUB = 2


def _gelu(x):
    return 0.5 * x * (1.0 + jax.lax.erf(x * (2.0 ** -0.5)))


def _dot(a, b):
    return jax.lax.dot_general(a, b, (((1,), (0,)), ((), ())),
                               preferred_element_type=jnp.float32)


def _dot_t(a, b):
    return jax.lax.dot_general(a, b, (((1,), (1,)), ((), ())),
                               preferred_element_type=jnp.float32)


def _vq_round(r, Cm2, cn):
    s = cn - 2.0 * _dot_t(r, Cm2)
    smin = jnp.min(s, axis=1, keepdims=True)
    oh = (s == smin).astype(jnp.float32)
    q = _dot(oh, Cm2)
    diff = r - q
    return diff, jnp.sum(diff * diff)


def _body(x_ref, w0_ref, b0_ref, w1_ref, b1_ref,
          dw0_ref, db0_ref, dw1_ref, db1_ref,
          cm0_ref, cm1_ref, cm2_ref,
          recon_ref, l0_ref, l1_ref, l2_ref,
          cn0_ref, cn1_ref, cn2_ref):
    i = pl.program_id(0)

    @pl.when(i == 0)
    def _init():
        recon_ref[0, 0] = 0.0
        l0_ref[0, 0] = 0.0
        l1_ref[0, 0] = 0.0
        l2_ref[0, 0] = 0.0
        for cref, nref in ((cm0_ref, cn0_ref), (cm1_ref, cn1_ref),
                           (cm2_ref, cn2_ref)):
            C = cref[...]
            nref[...] = jnp.sum(C * C, axis=1)[None, :]

    TB = x_ref.shape[0]
    SUB = TB // _NSUB
    sr = s0 = s1 = s2 = jnp.float32(0.0)
    for sub in range(_NSUB):
        x = x_ref[pl.ds(sub * SUB, SUB), :]
        h = _gelu(_dot(x, w0_ref[...]) + b0_ref[...])
        z = _dot(h, w1_ref[...]) + b1_ref[...]

        r = z
        r, a0 = _vq_round(r, cm0_ref[...], cn0_ref[...])
        r, a1 = _vq_round(r, cm1_ref[...], cn1_ref[...])
        r, a2 = _vq_round(r, cm2_ref[...], cn2_ref[...])
        zq = z - r

        h2 = _gelu(_dot(zq, dw0_ref[...]) + db0_ref[...])
        y = _dot(h2, dw1_ref[...]) + db1_ref[...]
        e = y - x
        sr += jnp.sum(e * e)
        s0, s1, s2 = s0 + a0, s1 + a1, s2 + a2

    recon_ref[0, 0] += sr
    l0_ref[0, 0] += s0
    l1_ref[0, 0] += s1
    l2_ref[0, 0] += s2


@functools.partial(jax.jit, static_argnames=())
def kernel(input, enc_W0, enc_b0, enc_W1, enc_b1,
           dec_W0, dec_b0, dec_W1, dec_b1, cb0, cb1, cb2):
    B, D = input.shape
    Z = enc_W1.shape[1]
    TB = 2048
    grid = B // TB

    full = lambda s: pl.BlockSpec(s, lambda i: (0,) * len(s))
    scal = pl.BlockSpec(memory_space=pltpu.SMEM)

    outs = pl.pallas_call(
        _body,
        grid=(grid,),
        in_specs=[
            pl.BlockSpec((TB, D), lambda i: (i, 0)),
            full(enc_W0.shape), full((1, enc_b0.shape[0])),
            full(enc_W1.shape), full((1, enc_b1.shape[0])),
            full(dec_W0.shape), full((1, dec_b0.shape[0])),
            full(dec_W1.shape), full((1, dec_b1.shape[0])),
            full(cb0.shape), full(cb1.shape), full(cb2.shape),
        ],
        out_specs=[scal, scal, scal, scal],
        out_shape=[jax.ShapeDtypeStruct((1, 1), jnp.float32)] * 4,
        scratch_shapes=[
            pltpu.VMEM((1, cb0.shape[0]), jnp.float32),
            pltpu.VMEM((1, cb1.shape[0]), jnp.float32),
            pltpu.VMEM((1, cb2.shape[0]), jnp.float32),
        ],
    )(input, enc_W0, enc_b0[None, :], enc_W1, enc_b1[None, :],
      dec_W0, dec_b0[None, :], dec_W1, dec_b1[None, :],
      cb0, cb1, cb2)

    recon_s, l0_s, l1_s, l2_s = outs
    recon = recon_s[0, 0] / (B * D)
    scale = (_BETA + _GAMMA) / (B * Z)
    return (recon, l0_s[0, 0] * scale, l1_s[0, 0] * scale, l2_s[0, 0] * scale)
